# baseline (speedup 1.0000x reference)
import functools

import jax
import jax.numpy as jnp
import numpy as np
from jax import lax
from jax.experimental import pallas as pl
from jax.experimental.pallas import tpu as pltpu
from jax.experimental.pallas import tpu_sc as plsc

N = 10000
D = 128
E = 320000
NLAB = 8

NCORES = 2
NSUB = 16
CH = 128
NCH = 2560
E_PAD = NCH * CH
NCH_T = NCH // NSUB
NPAD = 10112
RT = NPAD // NSUB
CW = 16
IB = 8
NB = NCH_T // IB

_PERM = np.empty((D,), np.int32)
for _g in range(D // 32):
    for _k in range(16):
        _PERM[_g * 32 + 2 * _k] = _g * 32 + _k
        _PERM[_g * 32 + 2 * _k + 1] = _g * 32 + 16 + _k


def _make_segsum(with_counts):
    mesh = plsc.VectorSubcoreMesh(
        core_axis_name="c", subcore_axis_name="s",
        num_cores=NCORES, num_subcores=NSUB)

    f32 = jnp.float32
    nf32 = 1 if with_counts else 2
    out_type = [jax.ShapeDtypeStruct((NPAD, D), f32),
                jax.ShapeDtypeStruct((NPAD, D), f32)]
    scratch = [
        pltpu.VMEM_SHARED((NPAD, D), f32),
        pltpu.VMEM((2, IB, CH), jnp.int32),
        pltpu.VMEM((2, IB, CH), jnp.int32),
        pltpu.VMEM((CH, D), jnp.bfloat16),
        pltpu.SemaphoreType.DMA,
        pltpu.SemaphoreType.DMA,
    ] + [pltpu.VMEM((CH, D), f32) for _ in range(nf32)] \
      + [pltpu.SemaphoreType.DMA for _ in range(nf32)]
    if with_counts:
        out_type += [jax.ShapeDtypeStruct((NPAD, CW), f32),
                     jax.ShapeDtypeStruct((NPAD, CW), f32)]
        scratch += [
            pltpu.VMEM_SHARED((NPAD, CW), f32),
            pltpu.VMEM((CH, CW), f32),
            pltpu.SemaphoreType.DMA,
        ]

    def body(*refs):
        if with_counts:
            (xa, xp, sap, dap, spa, dpa,
             sum_ap, sum_pa, cnt_ap, cnt_pa,
             acc, srcv, dstv, bfb, semg, semi,
             fb0, sems0, cacc, onesv, semc) = refs
            fbufs = (fb0,)
            sems = (sems0,)
        else:
            (xa, xp, sap, dap, spa, dpa,
             sum_ap, sum_pa,
             acc, srcv, dstv, bfb, semg, semi,
             fb0, fb1, sems0, sems1) = refs
            fbufs = (fb0, fb1)
            sems = (sems0, sems1)
            cnt_ap = cnt_pa = cacc = onesv = semc = None

        cid = lax.axis_index("c")
        wid = lax.axis_index("s")
        base = wid * RT
        cbase = wid * NCH_T

        def run(x_ref, s_ref, d_ref, sum_out, cnt_out):
            zv = jnp.zeros((16,), jnp.float32)
            fb0 = fbufs[0]

            def zrow(i, carry):
                def zcol(j, carry2):
                    fb0[i, pl.ds(j * 16, 16)] = zv
                    return carry2
                return lax.fori_loop(0, D // 16, zcol, carry)

            lax.fori_loop(0, CH, zrow, 0)
            for k in range(4):
                pltpu.sync_copy(fb0, acc.at[pl.ds(base + 128 * k, 128)])
            pltpu.sync_copy(fb0.at[pl.ds(0, RT - 512)],
                            acc.at[pl.ds(base + 512, RT - 512)])
            if with_counts:
                def z16row(i, carry):
                    onesv[i, :] = zv
                    return carry
                lax.fori_loop(0, CH, z16row, 0)
                for k in range(4):
                    pltpu.sync_copy(onesv,
                                    cacc.at[pl.ds(base + 128 * k, 128)])
                pltpu.sync_copy(onesv.at[pl.ds(0, RT - 512)],
                                cacc.at[pl.ds(base + 512, RT - 512)])
                onepat = jnp.where(
                    lax.iota(jnp.int32, 16) == 0, 1.0, 0.0
                ).astype(jnp.float32)

                def o16row(i, carry):
                    onesv[i, :] = onepat
                    return carry
                lax.fori_loop(0, CH, o16row, 0)
            plsc.subcore_barrier()

            def _wait_scatter(p):
                pltpu.make_async_copy(fbufs[p], acc.at[dstv.at[0, 0]],
                                      sems[p]).wait()

            def _wait_gather():
                pltpu.make_async_copy(x_ref.at[srcv.at[0, 0]], bfb,
                                      semg).wait()

            def _wait_idx():
                pltpu.make_async_copy(s_ref.at[pl.ds(cbase, IB)],
                                      srcv.at[0], semi).wait()

            def _wait_cnt():
                pltpu.make_async_copy(onesv, cacc.at[dstv.at[0, 0]],
                                      semc).wait()

            def _convert(fb):
                def crow(r, carry):
                    for g in range(D // 32):
                        v = bfb[r, pl.ds(g * 32, 32)]
                        lo, hi = plsc.unpack(
                            v, format=plsc.PackFormat.INTERLEAVED,
                            preferred_element_type=jnp.float32)
                        fb[r, pl.ds(g * 32, 16)] = lo
                        fb[r, pl.ds(g * 32 + 16, 16)] = hi
                    return carry
                lax.fori_loop(0, CH, crow, 0)

            pltpu.sync_copy(s_ref.at[pl.ds(cbase, IB)], srcv.at[0])
            pltpu.sync_copy(d_ref.at[pl.ds(cbase, IB)], dstv.at[0])
            pltpu.async_copy(x_ref.at[srcv.at[0, 0]], bfb, semg)

            def block(b, carry):
                par = lax.rem(b, 2)
                nxt = lax.rem(b + 1, 2)
                cbn = cbase + (b + 1) * IB

                @pl.when(b + 1 < NB)
                def _():
                    pltpu.async_copy(s_ref.at[pl.ds(cbn, IB)],
                                     srcv.at[nxt], semi)
                    pltpu.async_copy(d_ref.at[pl.ds(cbn, IB)],
                                     dstv.at[nxt], semi)

                for j in range(IB):
                    p = j % nf32
                    if j < nf32:
                        @pl.when(b > 0)
                        def _():
                            _wait_scatter(p)
                    else:
                        _wait_scatter(p)
                    _wait_gather()
                    _convert(fbufs[p])
                    if j + 1 < IB:
                        pltpu.async_copy(x_ref.at[srcv.at[par, j + 1]],
                                         bfb, semg)
                    else:
                        @pl.when(b + 1 < NB)
                        def _():
                            _wait_idx()
                            _wait_idx()
                            pltpu.async_copy(x_ref.at[srcv.at[nxt, 0]],
                                             bfb, semg)
                    pltpu.async_copy(fbufs[p], acc.at[dstv.at[par, j]],
                                     sems[p], add=True)
                    if with_counts:
                        pltpu.async_copy(onesv, cacc.at[dstv.at[par, j]],
                                         semc, add=True)
                        if j == 0:
                            @pl.when(b > 0)
                            def _():
                                _wait_cnt()
                        else:
                            _wait_cnt()
                return carry

            lax.fori_loop(0, NB, block, 0)
            for p in range(nf32):
                _wait_scatter(p)
            if with_counts:
                _wait_cnt()
            plsc.subcore_barrier()
            pltpu.sync_copy(acc.at[pl.ds(base, RT)],
                            sum_out.at[pl.ds(base, RT)])
            if with_counts:
                pltpu.sync_copy(cacc.at[pl.ds(base, RT)],
                                cnt_out.at[pl.ds(base, RT)])

        @pl.when(cid == 0)
        def _():
            run(xa, sap, dap, sum_ap, cnt_ap)

        @pl.when(cid == 1)
        def _():
            run(xp, spa, dpa, sum_pa, cnt_pa)

    return pl.kernel(body, out_type=out_type, mesh=mesh,
                     scratch_types=scratch,
                     compiler_params=pltpu.CompilerParams(
                         use_tc_tiling_on_sc=False,
                         needs_layout_passes=False),
                     name="segsum_l1" if with_counts else "segsum_l2")


_segsum_l1 = _make_segsum(True)
_segsum_l2 = _make_segsum(False)


def _dense_body(head, *refs):
    f32 = jnp.float32
    if head:
        (x_ref, s_ref, c_ref, Ws, Wd, Wu, bs, bd, bu, g, b, Wp, bp,
         o_ref) = refs
    else:
        (x_ref, s_ref, c_ref, Ws, Wd, Wu, bs, bd, bu, g, b, o_ref) = refs
    dot = functools.partial(jnp.dot, preferred_element_type=f32)
    Wu_d = Wu[0:D, :]
    Wu_s = Wu[D:2 * D, :]
    A = dot(Wd[...], Wu_d)
    B = dot(Ws[...], Wu_s)
    c = dot(bd[...], Wu_d) + dot(bs[...], Wu_s) + bu[...]
    cnt = jnp.maximum(c_ref[0:N, 0:1], 1.0)
    aggr = s_ref[0:N, :] / cnt
    h = dot(x_ref[...], A) + dot(aggr, B) + c
    mu = jnp.mean(h, axis=0, keepdims=True)
    dlt = h - mu
    var = jnp.mean(dlt * dlt, axis=0, keepdims=True)
    hn = dlt * lax.rsqrt(var + 1.0) * g[...] + b[...]
    act = jnp.where(hn >= 0.0, hn, 0.01 * hn)
    if head:
        o_ref[...] = dot(act, Wp[...]) + bp[...]
    else:
        o_ref[...] = act


def _dense(x, ssum, cnt, conv, bn, post=None):
    args = [x, ssum, cnt, conv["W_src"], conv["W_dst"], conv["W_upd"],
            conv["b_src"].reshape(1, D), conv["b_dst"].reshape(1, D),
            conv["b_upd"].reshape(1, D),
            bn["gamma"].reshape(1, D), bn["beta"].reshape(1, D)]
    if post is None:
        out = jax.ShapeDtypeStruct((N, D), jnp.float32)
    else:
        args += [post["W"], post["b"].reshape(1, NLAB)]
        out = jax.ShapeDtypeStruct((N, NLAB), jnp.float32)
    return pl.pallas_call(
        functools.partial(_dense_body, post is not None),
        out_shape=out)(*args)


def kernel(x_author, x_paper, edge_index_ap, edge_index_pa, params):
    i32 = jnp.int32
    f32 = jnp.float32
    ei_ap = edge_index_ap.astype(i32)
    ei_pa = edge_index_pa.astype(i32)
    pad_src = jnp.zeros((E_PAD - E,), i32)
    pad_dst = jnp.full((E_PAD - E,), N, i32)
    sap = jnp.concatenate([ei_ap[0], pad_src]).reshape(NCH, CH)
    dap = jnp.concatenate([ei_ap[1], pad_dst]).reshape(NCH, CH)
    spa = jnp.concatenate([ei_pa[0], pad_src]).reshape(NCH, CH)
    dpa = jnp.concatenate([ei_pa[1], pad_dst]).reshape(NCH, CH)
    xb_author = x_author.astype(jnp.bfloat16)[:, _PERM]
    xb_paper = x_paper.astype(jnp.bfloat16)[:, _PERM]
    sum_ap, sum_pa, cnt_ap, cnt_pa = _segsum_l1(
        xb_author, xb_paper, sap, dap, spa, dpa)

    p = params
    h_paper = _dense(x_paper, sum_ap, cnt_ap, p["conv1_ap"], p["bn1_paper"])
    h_author = _dense(x_author, sum_pa, cnt_pa, p["conv1_pa"], p["bn1_author"])

    hb_author = h_author.astype(jnp.bfloat16)[:, _PERM]
    hb_paper = h_paper.astype(jnp.bfloat16)[:, _PERM]
    sum2_ap, sum2_pa = _segsum_l2(
        hb_author, hb_paper, sap, dap, spa, dpa)

    out_paper = _dense(h_paper, sum2_ap, cnt_ap, p["conv2_ap"],
                       p["bn2_paper"], p["post_paper"])
    out_author = _dense(h_author, sum2_pa, cnt_pa, p["conv2_pa"],
                        p["bn2_author"], p["post_author"])
    return (out_author, out_paper)


# speedup vs baseline: 2.0229x; 2.0229x over previous
import functools

import jax
import jax.numpy as jnp
import numpy as np
from jax import lax
from jax.experimental import pallas as pl
from jax.experimental.pallas import tpu as pltpu
from jax.experimental.pallas import tpu_sc as plsc

N = 10000
D = 128
E = 320000
NLAB = 8

NCORES = 2
NSUB = 16
CH = 128
NCH = 2560
E_PAD = NCH * CH
NCH_T = NCH // NSUB
NPAD = 10112
RT = NPAD // NSUB
CW = 16
IB = 8
NB = NCH_T // IB

_PERM = np.empty((D,), np.int32)
for _g in range(D // 32):
    for _k in range(16):
        _PERM[_g * 32 + 2 * _k] = _g * 32 + _k
        _PERM[_g * 32 + 2 * _k + 1] = _g * 32 + 16 + _k


def _make_segsum(with_counts):
    mesh = plsc.VectorSubcoreMesh(
        core_axis_name="c", subcore_axis_name="s",
        num_cores=NCORES, num_subcores=NSUB)

    f32 = jnp.float32
    bf16 = jnp.bfloat16
    out_type = [jax.ShapeDtypeStruct((NPAD, D), bf16),
                jax.ShapeDtypeStruct((NPAD, D), bf16)]
    scratch = [
        pltpu.VMEM_SHARED((NPAD, D), bf16),
        pltpu.VMEM((2, IB, CH), jnp.int32),
        pltpu.VMEM((2, IB, CH), jnp.int32),
        pltpu.VMEM((CH, D), bf16),
        pltpu.VMEM((CH, D), bf16),
        pltpu.SemaphoreType.DMA,
        pltpu.SemaphoreType.DMA,
        pltpu.SemaphoreType.DMA,
        pltpu.SemaphoreType.DMA,
        pltpu.SemaphoreType.DMA,
    ]
    if with_counts:
        out_type += [jax.ShapeDtypeStruct((NPAD, CW), f32),
                     jax.ShapeDtypeStruct((NPAD, CW), f32)]
        scratch += [
            pltpu.VMEM_SHARED((NPAD, CW), f32),
            pltpu.VMEM((CH, CW), f32),
            pltpu.SemaphoreType.DMA,
        ]

    def body(*refs):
        if with_counts:
            (xa, xp, sap, dap, spa, dpa,
             sum_ap, sum_pa, cnt_ap, cnt_pa,
             acc, srcv, dstv, rows0, rows1,
             semg0, semg1, sems0, sems1, semi, cacc, onesv, semc) = refs
        else:
            (xa, xp, sap, dap, spa, dpa,
             sum_ap, sum_pa,
             acc, srcv, dstv, rows0, rows1,
             semg0, semg1, sems0, sems1, semi) = refs
            cnt_ap = cnt_pa = cacc = onesv = semc = None
        rows = (rows0, rows1)
        semg = (semg0, semg1)
        sems = (sems0, sems1)

        cid = lax.axis_index("c")
        wid = lax.axis_index("s")
        base = wid * RT
        cbase = wid * NCH_T

        def run(x_ref, s_ref, d_ref, sum_out, cnt_out):
            zv = jnp.zeros((16,), jnp.float32)
            zvb = jnp.zeros((32,), jnp.bfloat16)

            def zrow(i, carry):
                def zcol(j, carry2):
                    rows0[i, pl.ds(j * 32, 32)] = zvb
                    return carry2
                return lax.fori_loop(0, D // 32, zcol, carry)

            lax.fori_loop(0, CH, zrow, 0)
            for k in range(4):
                pltpu.sync_copy(rows0, acc.at[pl.ds(base + 128 * k, 128)])
            pltpu.sync_copy(rows0.at[pl.ds(0, RT - 512)],
                            acc.at[pl.ds(base + 512, RT - 512)])
            if with_counts:
                def z16row(i, carry):
                    onesv[i, :] = zv
                    return carry
                lax.fori_loop(0, CH, z16row, 0)
                for k in range(4):
                    pltpu.sync_copy(onesv,
                                    cacc.at[pl.ds(base + 128 * k, 128)])
                pltpu.sync_copy(onesv.at[pl.ds(0, RT - 512)],
                                cacc.at[pl.ds(base + 512, RT - 512)])
                onepat = jnp.where(
                    lax.iota(jnp.int32, 16) == 0, 1.0, 0.0
                ).astype(jnp.float32)

                def o16row(i, carry):
                    onesv[i, :] = onepat
                    return carry
                lax.fori_loop(0, CH, o16row, 0)
            plsc.subcore_barrier()

            def _wait_scatter(q):
                pltpu.make_async_copy(rows[q], acc.at[dstv.at[0, 0]],
                                      sems[q]).wait()

            def _wait_gather(p):
                pltpu.make_async_copy(x_ref.at[srcv.at[0, 0]], rows[p],
                                      semg[p]).wait()

            def _wait_idx():
                pltpu.make_async_copy(s_ref.at[pl.ds(cbase, IB)],
                                      srcv.at[0], semi).wait()

            def _wait_cnt():
                pltpu.make_async_copy(onesv, cacc.at[dstv.at[0, 0]],
                                      semc).wait()

            pltpu.sync_copy(s_ref.at[pl.ds(cbase, IB)], srcv.at[0])
            pltpu.sync_copy(d_ref.at[pl.ds(cbase, IB)], dstv.at[0])
            pltpu.async_copy(x_ref.at[srcv.at[0, 0]], rows[0], semg[0])

            def block(b, carry):
                par = lax.rem(b, 2)
                nxt = lax.rem(b + 1, 2)
                cbn = cbase + (b + 1) * IB

                @pl.when(b + 1 < NB)
                def _():
                    pltpu.async_copy(s_ref.at[pl.ds(cbn, IB)],
                                     srcv.at[nxt], semi)
                    pltpu.async_copy(d_ref.at[pl.ds(cbn, IB)],
                                     dstv.at[nxt], semi)

                for j in range(IB):
                    p = j % 2
                    q = 1 - p
                    if j == 0:
                        @pl.when(b > 0)
                        def _():
                            _wait_scatter(q)
                    else:
                        _wait_scatter(q)
                    if j + 1 < IB:
                        pltpu.async_copy(x_ref.at[srcv.at[par, j + 1]],
                                         rows[q], semg[q])
                    else:
                        @pl.when(b + 1 < NB)
                        def _():
                            _wait_idx()
                            _wait_idx()
                            pltpu.async_copy(x_ref.at[srcv.at[nxt, 0]],
                                             rows[q], semg[q])
                    _wait_gather(p)
                    pltpu.async_copy(rows[p], acc.at[dstv.at[par, j]],
                                     sems[p], add=True)
                    if with_counts:
                        pltpu.async_copy(onesv, cacc.at[dstv.at[par, j]],
                                         semc, add=True)
                        if j == 0:
                            @pl.when(b > 0)
                            def _():
                                _wait_cnt()
                        else:
                            _wait_cnt()
                return carry

            lax.fori_loop(0, NB, block, 0)
            _wait_scatter(1)
            if with_counts:
                _wait_cnt()
            plsc.subcore_barrier()
            pltpu.sync_copy(acc.at[pl.ds(base, RT)],
                            sum_out.at[pl.ds(base, RT)])
            if with_counts:
                pltpu.sync_copy(cacc.at[pl.ds(base, RT)],
                                cnt_out.at[pl.ds(base, RT)])

        @pl.when(cid == 0)
        def _():
            run(xa, sap, dap, sum_ap, cnt_ap)

        @pl.when(cid == 1)
        def _():
            run(xp, spa, dpa, sum_pa, cnt_pa)

    return pl.kernel(body, out_type=out_type, mesh=mesh,
                     scratch_types=scratch,
                     compiler_params=pltpu.CompilerParams(
                         use_tc_tiling_on_sc=False),
                     name="segsum_l1" if with_counts else "segsum_l2")


_segsum_l1 = _make_segsum(True)
_segsum_l2 = _make_segsum(False)


def _dense_body(head, *refs):
    f32 = jnp.float32
    if head:
        (x_ref, s_ref, c_ref, Ws, Wd, Wu, bs, bd, bu, g, b, Wp, bp,
         o_ref) = refs
    else:
        (x_ref, s_ref, c_ref, Ws, Wd, Wu, bs, bd, bu, g, b, o_ref) = refs
    dot = functools.partial(jnp.dot, preferred_element_type=f32)
    Wu_d = Wu[0:D, :]
    Wu_s = Wu[D:2 * D, :]
    A = dot(Wd[...], Wu_d)
    B = dot(Ws[...], Wu_s)
    c = dot(bd[...], Wu_d) + dot(bs[...], Wu_s) + bu[...]
    cnt = jnp.maximum(c_ref[0:N, 0:1], 1.0)
    aggr = s_ref[0:N, :].astype(f32) / cnt
    h = dot(x_ref[...], A) + dot(aggr, B) + c
    mu = jnp.mean(h, axis=0, keepdims=True)
    dlt = h - mu
    var = jnp.mean(dlt * dlt, axis=0, keepdims=True)
    hn = dlt * lax.rsqrt(var + 1.0) * g[...] + b[...]
    act = jnp.where(hn >= 0.0, hn, 0.01 * hn)
    if head:
        o_ref[...] = dot(act, Wp[...]) + bp[...]
    else:
        o_ref[...] = act


def _dense(x, ssum, cnt, conv, bn, post=None):
    args = [x, ssum, cnt, conv["W_src"][_PERM], conv["W_dst"], conv["W_upd"],
            conv["b_src"].reshape(1, D), conv["b_dst"].reshape(1, D),
            conv["b_upd"].reshape(1, D),
            bn["gamma"].reshape(1, D), bn["beta"].reshape(1, D)]
    if post is None:
        out = jax.ShapeDtypeStruct((N, D), jnp.float32)
    else:
        args += [post["W"], post["b"].reshape(1, NLAB)]
        out = jax.ShapeDtypeStruct((N, NLAB), jnp.float32)
    return pl.pallas_call(
        functools.partial(_dense_body, post is not None),
        out_shape=out)(*args)


def kernel(x_author, x_paper, edge_index_ap, edge_index_pa, params):
    i32 = jnp.int32
    f32 = jnp.float32
    ei_ap = edge_index_ap.astype(i32)
    ei_pa = edge_index_pa.astype(i32)
    pad_src = jnp.zeros((E_PAD - E,), i32)
    pad_dst = jnp.full((E_PAD - E,), N, i32)
    sap = jnp.concatenate([ei_ap[0], pad_src]).reshape(NCH, CH)
    dap = jnp.concatenate([ei_ap[1], pad_dst]).reshape(NCH, CH)
    spa = jnp.concatenate([ei_pa[0], pad_src]).reshape(NCH, CH)
    dpa = jnp.concatenate([ei_pa[1], pad_dst]).reshape(NCH, CH)
    xb_author = x_author.astype(jnp.bfloat16)[:, _PERM]
    xb_paper = x_paper.astype(jnp.bfloat16)[:, _PERM]
    sum_ap, sum_pa, cnt_ap, cnt_pa = _segsum_l1(
        xb_author, xb_paper, sap, dap, spa, dpa)

    p = params
    h_paper = _dense(x_paper, sum_ap, cnt_ap, p["conv1_ap"], p["bn1_paper"])
    h_author = _dense(x_author, sum_pa, cnt_pa, p["conv1_pa"], p["bn1_author"])

    hb_author = h_author.astype(jnp.bfloat16)[:, _PERM]
    hb_paper = h_paper.astype(jnp.bfloat16)[:, _PERM]
    sum2_ap, sum2_pa = _segsum_l2(
        hb_author, hb_paper, sap, dap, spa, dpa)

    out_paper = _dense(h_paper, sum2_ap, cnt_ap, p["conv2_ap"],
                       p["bn2_paper"], p["post_paper"])
    out_author = _dense(h_author, sum2_pa, cnt_pa, p["conv2_pa"],
                        p["bn2_author"], p["post_author"])
    return (out_author, out_paper)


# speedup vs baseline: 2.0631x; 1.0199x over previous
import functools

import jax
import jax.numpy as jnp
import numpy as np
from jax import lax
from jax.experimental import pallas as pl
from jax.experimental.pallas import tpu as pltpu
from jax.experimental.pallas import tpu_sc as plsc

N = 10000
D = 128
E = 320000
NLAB = 8

NCORES = 2
NSUB = 16
CH = 128
NCH = 2560
E_PAD = NCH * CH
NCH_T = NCH // NSUB
NPAD = 10112
RT = NPAD // NSUB
CW = 16
IB = 8
NB = NCH_T // IB

_PERM = np.empty((D,), np.int32)
for _g in range(D // 32):
    for _k in range(16):
        _PERM[_g * 32 + 2 * _k] = _g * 32 + _k
        _PERM[_g * 32 + 2 * _k + 1] = _g * 32 + 16 + _k


def _make_segsum(with_counts):
    mesh = plsc.VectorSubcoreMesh(
        core_axis_name="c", subcore_axis_name="s",
        num_cores=NCORES, num_subcores=NSUB)

    f32 = jnp.float32
    bf16 = jnp.bfloat16
    out_type = [jax.ShapeDtypeStruct((NPAD, D), bf16),
                jax.ShapeDtypeStruct((NPAD, D), bf16)]
    scratch = [
        pltpu.VMEM_SHARED((NPAD, D), bf16),
        pltpu.VMEM((2, IB, CH), jnp.int32),
        pltpu.VMEM((2, IB, CH), jnp.int32),
        pltpu.VMEM((CH, D), bf16),
        pltpu.VMEM((CH, D), bf16),
        pltpu.VMEM((CH, D), bf16),
        pltpu.VMEM((CH, D), bf16),
        pltpu.SemaphoreType.DMA,
        pltpu.SemaphoreType.DMA,
        pltpu.SemaphoreType.DMA,
        pltpu.SemaphoreType.DMA,
        pltpu.SemaphoreType.DMA,
        pltpu.SemaphoreType.DMA,
        pltpu.SemaphoreType.DMA,
        pltpu.SemaphoreType.DMA,
        pltpu.SemaphoreType.DMA,
    ]
    if with_counts:
        out_type += [jax.ShapeDtypeStruct((NPAD, CW), f32),
                     jax.ShapeDtypeStruct((NPAD, CW), f32)]
        scratch += [
            pltpu.VMEM_SHARED((NPAD, CW), f32),
            pltpu.VMEM((CH, CW), f32),
            pltpu.SemaphoreType.DMA,
        ]

    def body(*refs):
        if with_counts:
            (xa, xp, sap, dap, spa, dpa,
             sum_ap, sum_pa, cnt_ap, cnt_pa,
             acc, srcv, dstv, rows0, rows1, rows2, rows3,
             semg0, semg1, semg2, semg3,
             sems0, sems1, sems2, sems3, semi, cacc, onesv, semc) = refs
        else:
            (xa, xp, sap, dap, spa, dpa,
             sum_ap, sum_pa,
             acc, srcv, dstv, rows0, rows1, rows2, rows3,
             semg0, semg1, semg2, semg3,
             sems0, sems1, sems2, sems3, semi) = refs
            cnt_ap = cnt_pa = cacc = onesv = semc = None
        rows = (rows0, rows1, rows2, rows3)
        semg = (semg0, semg1, semg2, semg3)
        sems = (sems0, sems1, sems2, sems3)

        cid = lax.axis_index("c")
        wid = lax.axis_index("s")
        base = wid * RT
        cbase = wid * NCH_T

        def run(x_ref, s_ref, d_ref, sum_out, cnt_out):
            zv = jnp.zeros((16,), jnp.float32)
            zvb = jnp.zeros((32,), jnp.bfloat16)

            def zrow(i, carry):
                def zcol(j, carry2):
                    rows0[i, pl.ds(j * 32, 32)] = zvb
                    return carry2
                return lax.fori_loop(0, D // 32, zcol, carry)

            lax.fori_loop(0, CH, zrow, 0)
            for k in range(4):
                pltpu.sync_copy(rows0, acc.at[pl.ds(base + 128 * k, 128)])
            pltpu.sync_copy(rows0.at[pl.ds(0, RT - 512)],
                            acc.at[pl.ds(base + 512, RT - 512)])
            if with_counts:
                def z16row(i, carry):
                    onesv[i, :] = zv
                    return carry
                lax.fori_loop(0, CH, z16row, 0)
                for k in range(4):
                    pltpu.sync_copy(onesv,
                                    cacc.at[pl.ds(base + 128 * k, 128)])
                pltpu.sync_copy(onesv.at[pl.ds(0, RT - 512)],
                                cacc.at[pl.ds(base + 512, RT - 512)])
                onepat = jnp.where(
                    lax.iota(jnp.int32, 16) == 0, 1.0, 0.0
                ).astype(jnp.float32)

                def o16row(i, carry):
                    onesv[i, :] = onepat
                    return carry
                lax.fori_loop(0, CH, o16row, 0)
            plsc.subcore_barrier()

            def _wait_scatter(q):
                pltpu.make_async_copy(rows[q], acc.at[dstv.at[0, 0]],
                                      sems[q]).wait()

            def _wait_gather(p):
                pltpu.make_async_copy(x_ref.at[srcv.at[0, 0]], rows[p],
                                      semg[p]).wait()

            def _wait_idx():
                pltpu.make_async_copy(s_ref.at[pl.ds(cbase, IB)],
                                      srcv.at[0], semi).wait()

            def _wait_cnt():
                pltpu.make_async_copy(onesv, cacc.at[dstv.at[0, 0]],
                                      semc).wait()

            pltpu.sync_copy(s_ref.at[pl.ds(cbase, IB)], srcv.at[0])
            pltpu.sync_copy(d_ref.at[pl.ds(cbase, IB)], dstv.at[0])
            pltpu.async_copy(x_ref.at[srcv.at[0, 0]], rows[0], semg[0])
            pltpu.async_copy(x_ref.at[srcv.at[0, 1]], rows[1], semg[1])

            def block(b, carry):
                par = lax.rem(b, 2)
                nxt = lax.rem(b + 1, 2)
                cbn = cbase + (b + 1) * IB

                @pl.when(b + 1 < NB)
                def _():
                    pltpu.async_copy(s_ref.at[pl.ds(cbn, IB)],
                                     srcv.at[nxt], semi)
                    pltpu.async_copy(d_ref.at[pl.ds(cbn, IB)],
                                     dstv.at[nxt], semi)

                for j in range(IB):
                    p = j % 4
                    f = (j + 2) % 4
                    if j < 2:
                        @pl.when(b > 0)
                        def _():
                            _wait_scatter(f)
                    else:
                        _wait_scatter(f)
                    if j + 2 < IB:
                        pltpu.async_copy(x_ref.at[srcv.at[par, j + 2]],
                                         rows[f], semg[f])
                    else:
                        @pl.when(b + 1 < NB)
                        def _():
                            if j + 2 == IB:
                                _wait_idx()
                                _wait_idx()
                            pltpu.async_copy(
                                x_ref.at[srcv.at[nxt, j + 2 - IB]],
                                rows[f], semg[f])
                    _wait_gather(p)
                    pltpu.async_copy(rows[p], acc.at[dstv.at[par, j]],
                                     sems[p], add=True)
                    if with_counts:
                        pltpu.async_copy(onesv, cacc.at[dstv.at[par, j]],
                                         semc, add=True)
                        if j == 0:
                            @pl.when(b > 0)
                            def _():
                                _wait_cnt()
                        else:
                            _wait_cnt()
                return carry

            lax.fori_loop(0, NB, block, 0)
            _wait_scatter((NCH_T - 2) % 4)
            _wait_scatter((NCH_T - 1) % 4)
            if with_counts:
                _wait_cnt()
            plsc.subcore_barrier()
            pltpu.sync_copy(acc.at[pl.ds(base, RT)],
                            sum_out.at[pl.ds(base, RT)])
            if with_counts:
                pltpu.sync_copy(cacc.at[pl.ds(base, RT)],
                                cnt_out.at[pl.ds(base, RT)])

        @pl.when(cid == 0)
        def _():
            run(xa, sap, dap, sum_ap, cnt_ap)

        @pl.when(cid == 1)
        def _():
            run(xp, spa, dpa, sum_pa, cnt_pa)

    return pl.kernel(body, out_type=out_type, mesh=mesh,
                     scratch_types=scratch,
                     compiler_params=pltpu.CompilerParams(
                         use_tc_tiling_on_sc=False),
                     name="segsum_l1" if with_counts else "segsum_l2")


_segsum_l1 = _make_segsum(True)
_segsum_l2 = _make_segsum(False)


def _dense_body(head, *refs):
    f32 = jnp.float32
    if head:
        (x_ref, s_ref, c_ref, Ws, Wd, Wu, bs, bd, bu, g, b, Wp, bp,
         o_ref) = refs
    else:
        (x_ref, s_ref, c_ref, Ws, Wd, Wu, bs, bd, bu, g, b, o_ref) = refs
    dot = functools.partial(jnp.dot, preferred_element_type=f32)
    Wu_d = Wu[0:D, :]
    Wu_s = Wu[D:2 * D, :]
    A = dot(Wd[...], Wu_d)
    B = dot(Ws[...], Wu_s)
    c = dot(bd[...], Wu_d) + dot(bs[...], Wu_s) + bu[...]
    cnt = jnp.maximum(c_ref[0:N, 0:1], 1.0)
    aggr = s_ref[0:N, :].astype(f32) / cnt
    h = dot(x_ref[...], A) + dot(aggr, B) + c
    mu = jnp.mean(h, axis=0, keepdims=True)
    dlt = h - mu
    var = jnp.mean(dlt * dlt, axis=0, keepdims=True)
    hn = dlt * lax.rsqrt(var + 1.0) * g[...] + b[...]
    act = jnp.where(hn >= 0.0, hn, 0.01 * hn)
    if head:
        o_ref[...] = dot(act, Wp[...]) + bp[...]
    else:
        o_ref[...] = act


def _dense(x, ssum, cnt, conv, bn, post=None):
    args = [x, ssum, cnt, conv["W_src"][_PERM], conv["W_dst"], conv["W_upd"],
            conv["b_src"].reshape(1, D), conv["b_dst"].reshape(1, D),
            conv["b_upd"].reshape(1, D),
            bn["gamma"].reshape(1, D), bn["beta"].reshape(1, D)]
    if post is None:
        out = jax.ShapeDtypeStruct((N, D), jnp.float32)
    else:
        args += [post["W"], post["b"].reshape(1, NLAB)]
        out = jax.ShapeDtypeStruct((N, NLAB), jnp.float32)
    return pl.pallas_call(
        functools.partial(_dense_body, post is not None),
        out_shape=out)(*args)


def kernel(x_author, x_paper, edge_index_ap, edge_index_pa, params):
    i32 = jnp.int32
    f32 = jnp.float32
    ei_ap = edge_index_ap.astype(i32)
    ei_pa = edge_index_pa.astype(i32)
    pad_src = jnp.zeros((E_PAD - E,), i32)
    pad_dst = jnp.full((E_PAD - E,), N, i32)
    sap = jnp.concatenate([ei_ap[0], pad_src]).reshape(NCH, CH)
    dap = jnp.concatenate([ei_ap[1], pad_dst]).reshape(NCH, CH)
    spa = jnp.concatenate([ei_pa[0], pad_src]).reshape(NCH, CH)
    dpa = jnp.concatenate([ei_pa[1], pad_dst]).reshape(NCH, CH)
    xb_author = x_author.astype(jnp.bfloat16)[:, _PERM]
    xb_paper = x_paper.astype(jnp.bfloat16)[:, _PERM]
    sum_ap, sum_pa, cnt_ap, cnt_pa = _segsum_l1(
        xb_author, xb_paper, sap, dap, spa, dpa)

    p = params
    h_paper = _dense(x_paper, sum_ap, cnt_ap, p["conv1_ap"], p["bn1_paper"])
    h_author = _dense(x_author, sum_pa, cnt_pa, p["conv1_pa"], p["bn1_author"])

    hb_author = h_author.astype(jnp.bfloat16)[:, _PERM]
    hb_paper = h_paper.astype(jnp.bfloat16)[:, _PERM]
    sum2_ap, sum2_pa = _segsum_l2(
        hb_author, hb_paper, sap, dap, spa, dpa)

    out_paper = _dense(h_paper, sum2_ap, cnt_ap, p["conv2_ap"],
                       p["bn2_paper"], p["post_paper"])
    out_author = _dense(h_author, sum2_pa, cnt_pa, p["conv2_pa"],
                        p["bn2_author"], p["post_author"])
    return (out_author, out_paper)
